# Initial kernel scaffold; baseline (speedup 1.0000x reference)
import jax
import jax.numpy as jnp
from jax.experimental import pallas as pl


def kernel(x, router_w, router_b, fc1_w, fc1_b, gate, fc2_w, fc2_b, sfc1_w, sfc1_b, sgate, sfc2_w, sfc2_b, shared_weight):
    raise NotImplementedError("write your pallas kernel here")


# speedup vs baseline: 3.3462x; 3.3462x over previous
import functools
import jax
import jax.numpy as jnp
from jax.experimental import pallas as pl
from jax.experimental.pallas import tpu as pltpu

S, D, F, E, K = 2048, 1024, 4096, 8, 2
FCH = 512
NF = F // FCH


def _gelu(v):
    return 0.5 * v * (1.0 + jax.lax.erf(v * 0.7071067811865476))


def _top2_combine(logits):
    ids = jax.lax.broadcasted_iota(jnp.int32, logits.shape, 1)
    m1 = jnp.max(logits, axis=1, keepdims=True)
    is1 = logits == m1
    i1 = jnp.min(jnp.where(is1, ids, E), axis=1, keepdims=True)
    sel1 = ids == i1
    l1 = jnp.sum(jnp.where(sel1, logits, 0.0), axis=1, keepdims=True)
    neg = jnp.full_like(logits, -jnp.inf)
    lm = jnp.where(sel1, neg, logits)
    m2 = jnp.max(lm, axis=1, keepdims=True)
    is2 = lm == m2
    i2 = jnp.min(jnp.where(is2, ids, E), axis=1, keepdims=True)
    sel2 = ids == i2
    l2 = jnp.sum(jnp.where(sel2, lm, 0.0), axis=1, keepdims=True)
    w1 = jax.nn.sigmoid(l1 - l2)
    w2 = 1.0 - w1
    return jnp.where(sel1, w1, 0.0) + jnp.where(sel2, w2, 0.0)


def _moe_body(x_ref, rw_ref, rb_ref, w1_ref, b1_ref, g_ref, w2_ref, b2_ref,
              out_ref, comb_ref, acc_ref):
    e = pl.program_id(0)
    f = pl.program_id(1)

    @pl.when((e == 0) & (f == 0))
    def _():
        logits = jax.lax.dot_general(
            x_ref[...], rw_ref[...], (((1,), (1,)), ((), ())),
            preferred_element_type=jnp.float32) + rb_ref[...][None, :]
        comb_ref[...] = _top2_combine(logits)
        acc_ref[...] = jnp.zeros_like(acc_ref)

    onehot = (jax.lax.broadcasted_iota(jnp.int32, (E, 1), 0) == e
              ).astype(jnp.float32)
    c = jax.lax.dot_general(
        comb_ref[...], onehot, (((1,), (0,)), ((), ())),
        preferred_element_type=jnp.float32)
    h = jax.lax.dot_general(
        x_ref[...], w1_ref[0], (((1,), (1,)), ((), ())),
        preferred_element_type=jnp.float32) + b1_ref[0]
    h = _gelu(h * g_ref[0])
    acc_ref[...] += jax.lax.dot_general(
        c * h, w2_ref[0], (((1,), (1,)), ((), ())),
        preferred_element_type=jnp.float32)

    @pl.when(f == 0)
    def _():
        acc_ref[...] += c * b2_ref[0]

    @pl.when((e == E - 1) & (f == NF - 1))
    def _():
        out_ref[...] = acc_ref[...]


def _shared_body(x_ref, mo_ref, w1_ref, b1_ref, g_ref, w2_ref, b2_ref,
                 sw_ref, out_ref, acc_ref):
    f = pl.program_id(0)

    @pl.when(f == 0)
    def _():
        acc_ref[...] = mo_ref[...] + jax.nn.sigmoid(sw_ref[0]) * b2_ref[...][None, :]

    h = jax.lax.dot_general(
        x_ref[...], w1_ref[...], (((1,), (1,)), ((), ())),
        preferred_element_type=jnp.float32) + b1_ref[...][None, :]
    h = _gelu(h * g_ref[...][None, :])
    acc_ref[...] += jax.nn.sigmoid(sw_ref[0]) * jax.lax.dot_general(
        h, w2_ref[...], (((1,), (1,)), ((), ())),
        preferred_element_type=jnp.float32)

    @pl.when(f == NF - 1)
    def _():
        out_ref[...] = acc_ref[...]


def kernel(x, router_w, router_b, fc1_w, fc1_b, gate, fc2_w, fc2_b,
           sfc1_w, sfc1_b, sgate, sfc2_w, sfc2_b, shared_weight):
    xs = x.reshape(S, D)

    moe_out = pl.pallas_call(
        _moe_body,
        grid=(E, NF),
        in_specs=[
            pl.BlockSpec((S, D), lambda e, f: (0, 0)),
            pl.BlockSpec((E, D), lambda e, f: (0, 0)),
            pl.BlockSpec((E,), lambda e, f: (0,)),
            pl.BlockSpec((1, FCH, D), lambda e, f: (e, f, 0)),
            pl.BlockSpec((1, 1, FCH), lambda e, f: (e * NF + f, 0, 0)),
            pl.BlockSpec((1, 1, FCH), lambda e, f: (e * NF + f, 0, 0)),
            pl.BlockSpec((1, D, FCH), lambda e, f: (e, 0, f)),
            pl.BlockSpec((1, 1, D), lambda e, f: (e, 0, 0)),
        ],
        out_specs=pl.BlockSpec((S, D), lambda e, f: (0, 0)),
        out_shape=jax.ShapeDtypeStruct((S, D), jnp.float32),
        scratch_shapes=[
            pltpu.VMEM((S, E), jnp.float32),
            pltpu.VMEM((S, D), jnp.float32),
        ],
    )(xs, router_w, router_b, fc1_w,
      fc1_b.reshape(E * NF, 1, FCH), gate.reshape(E * NF, 1, FCH),
      fc2_w, fc2_b.reshape(E, 1, D))

    out = pl.pallas_call(
        _shared_body,
        grid=(NF,),
        in_specs=[
            pl.BlockSpec((S, D), lambda f: (0, 0)),
            pl.BlockSpec((S, D), lambda f: (0, 0)),
            pl.BlockSpec((FCH, D), lambda f: (f, 0)),
            pl.BlockSpec((FCH,), lambda f: (f,)),
            pl.BlockSpec((FCH,), lambda f: (f,)),
            pl.BlockSpec((D, FCH), lambda f: (0, f)),
            pl.BlockSpec((D,), lambda f: (0,)),
            pl.BlockSpec(memory_space=pltpu.SMEM),
        ],
        out_specs=pl.BlockSpec((S, D), lambda f: (0, 0)),
        out_shape=jax.ShapeDtypeStruct((S, D), jnp.float32),
        scratch_shapes=[pltpu.VMEM((S, D), jnp.float32)],
    )(xs, moe_out, sfc1_w, sfc1_b, sgate, sfc2_w, sfc2_b,
      shared_weight.reshape(1))

    return out.reshape(1, S, D)
